# baseline (speedup 1.0000x reference)
import functools

import jax
import jax.numpy as jnp
from jax import lax
from jax.experimental import pallas as pl
from jax.experimental.pallas import tpu as pltpu
from jax.experimental.pallas import tpu_sc as plsc

HIDDEN = 64
NUM_WORKERS = 32
CHUNK = 800


def _emb_lookup_sc(idx_flat, emb, n):
    bpw = n // NUM_WORKERS
    nchunks = bpw // CHUNK
    mesh = plsc.VectorSubcoreMesh(core_axis_name="c", subcore_axis_name="s")

    @functools.partial(
        pl.kernel,
        mesh=mesh,
        out_type=jax.ShapeDtypeStruct((n, HIDDEN), jnp.float32),
        compiler_params=pltpu.CompilerParams(use_tc_tiling_on_sc=False),
        scratch_types=[
            pltpu.VMEM((bpw,), jnp.int32),
            pltpu.VMEM((CHUNK, HIDDEN), jnp.float32),
            pltpu.VMEM((CHUNK, HIDDEN), jnp.float32),
            pltpu.SemaphoreType.DMA,
            pltpu.SemaphoreType.DMA,
        ],
    )
    def k(idx_hbm, table_hbm, out_hbm, idx_v, buf0, buf1, gsem, wsem):
        wid = lax.axis_index("s") * 2 + lax.axis_index("c")
        base = wid * bpw
        pltpu.sync_copy(idx_hbm.at[pl.ds(base, bpw)], idx_v)

        bufs = (buf0, buf1)
        gathers = [None] * nchunks
        writes = [None] * nchunks

        gathers[0] = pltpu.async_copy(
            table_hbm.at[idx_v.at[pl.ds(0, CHUNK)]], bufs[0], gsem)
        for g in range(nchunks):
            gathers[g].wait()
            if g >= 1:
                writes[g - 1].wait()
            if g + 1 < nchunks:
                gathers[g + 1] = pltpu.async_copy(
                    table_hbm.at[idx_v.at[pl.ds((g + 1) * CHUNK, CHUNK)]],
                    bufs[(g + 1) % 2], gsem)
            writes[g] = pltpu.async_copy(
                bufs[g % 2], out_hbm.at[pl.ds(base + g * CHUNK, CHUNK)], wsem)
        writes[nchunks - 1].wait()

    return k(idx_flat, emb)


def kernel(x, emb):
    b, l = x.shape
    idx_flat = x.reshape(-1).astype(jnp.int32)
    out = _emb_lookup_sc(idx_flat, emb, b * l)
    return out.reshape(b, l, HIDDEN)


# speedup vs baseline: 1.0003x; 1.0003x over previous
import functools

import jax
import jax.numpy as jnp
from jax import lax
from jax.experimental import pallas as pl
from jax.experimental.pallas import tpu as pltpu
from jax.experimental.pallas import tpu_sc as plsc

HIDDEN = 64
NUM_WORKERS = 32
CHUNK = 800


def _emb_lookup_sc(idx_flat, emb, b, l):
    n = b * l
    bpw = n // NUM_WORKERS
    nchunks = bpw // CHUNK
    rows_per_chunk = CHUNK // l
    mesh = plsc.VectorSubcoreMesh(core_axis_name="c", subcore_axis_name="s")

    @functools.partial(
        pl.kernel,
        mesh=mesh,
        out_type=jax.ShapeDtypeStruct((b, l, HIDDEN), jnp.float32),
        compiler_params=pltpu.CompilerParams(use_tc_tiling_on_sc=False),
        scratch_types=[
            pltpu.VMEM((bpw,), jnp.int32),
            pltpu.VMEM((CHUNK, HIDDEN), jnp.float32),
            pltpu.VMEM((CHUNK, HIDDEN), jnp.float32),
            pltpu.SemaphoreType.DMA,
            pltpu.SemaphoreType.DMA,
        ],
    )
    def k(idx_hbm, table_hbm, out_hbm, idx_v, buf0, buf1, gsem, wsem):
        wid = lax.axis_index("s") * 2 + lax.axis_index("c")
        base = wid * bpw
        pltpu.sync_copy(idx_hbm.at[pl.ds(base, bpw)], idx_v)

        bufs = (buf0, buf1)
        gathers = [None] * nchunks
        writes = [None] * nchunks

        gathers[0] = pltpu.async_copy(
            table_hbm.at[idx_v.at[pl.ds(0, CHUNK)]], bufs[0], gsem)
        for g in range(nchunks):
            gathers[g].wait()
            if g >= 1:
                for w in writes[g - 1]:
                    w.wait()
            if g + 1 < nchunks:
                gathers[g + 1] = pltpu.async_copy(
                    table_hbm.at[idx_v.at[pl.ds((g + 1) * CHUNK, CHUNK)]],
                    bufs[(g + 1) % 2], gsem)
            b0 = (base + g * CHUNK) // l
            writes[g] = [
                pltpu.async_copy(
                    bufs[g % 2].at[pl.ds(j * l, l)], out_hbm.at[b0 + j], wsem)
                for j in range(rows_per_chunk)
            ]
        for w in writes[nchunks - 1]:
            w.wait()

    return k(idx_flat, emb)


def kernel(x, emb):
    b, l = x.shape
    idx_flat = x.reshape(-1).astype(jnp.int32)
    return _emb_lookup_sc(idx_flat, emb, b, l)
